# baseline (speedup 1.0000x reference)
import functools

import jax
import jax.numpy as jnp
from jax import lax
from jax.experimental import pallas as pl
from jax.experimental.pallas import tpu as pltpu
from jax.experimental.pallas import tpu_sc as plsc

N = 10000
NP = 10240
ND = NP
NA = 10016
E = 320000
ET = E + N
NC = 2
NS = 16
NW = NC * NS
BE = 48
NBG = 24
NG = 9
NG0 = 10
NG1 = 8
EPW = NG * NBG * BE
EP = NW * EPW
RPW = NP // NS
NEG = 0.2
D = 128

_mesh = plsc.VectorSubcoreMesh(
    core_axis_name="c", subcore_axis_name="s", num_cores=NC, num_subcores=NS
)


def _sc_edge_body(src_hbm, dst_hbm, asrc_hbm, adst_hbm, m_hbm, h_hbm,
                  out_hbm, den_hbm,
                  src_g, dst_g, ee_g, asrc_v, adst_v, zrow, m_v,
                  buf0, buf1, buf2, out_sh, den_sh,
                  g0, g1, g2, s0, s1, s2, d0):
    cid = lax.axis_index("c")
    sid = lax.axis_index("s")
    wid = cid * NS + sid
    bufs = (buf0, buf1, buf2)
    gsem = (g0, g1, g2)
    ssem = (s0, s1, s2)

    ds0 = pltpu.async_copy(asrc_hbm, asrc_v, g0)
    ds1 = pltpu.async_copy(adst_hbm, adst_v, g1)
    ds2 = pltpu.async_copy(m_hbm, m_v, g2)

    zero16 = jnp.zeros((16,), jnp.float32)
    for q in range(8):
        zrow[pl.ds(q * 16, 16)] = zero16
    ds0.wait()
    ds1.wait()
    ds2.wait()

    def _zh(i, _):
        for q in range(8):
            buf0[i, pl.ds(q * 16, 16)] = zero16
        return 0
    lax.fori_loop(0, BE, _zh, 0)

    base = sid * RPW
    zds = []
    for t in range(13):
        zds.append(pltpu.async_copy(buf0, out_sh.at[pl.ds(base + t * BE, BE)],
                                    g1))
    zds.append(pltpu.async_copy(buf0.at[pl.ds(0, 16)],
                                out_sh.at[pl.ds(base + 624, 16)], g1))
    for t in range(RPW // 128):
        zds.append(pltpu.async_copy(zrow, den_sh.at[pl.ds(base + t * 128, 128)],
                                    g1))
    for zd in zds:
        zd.wait()
    plsc.subcore_barrier()

    mvec = m_v[...]

    ngrp = jnp.where(cid == 0, NG0, NG1)
    rowb = jnp.where(cid == 0, sid * NG0, NS * NG0 + sid * NG1)

    def _grp(g, _):
        row = rowb + g
        pltpu.sync_copy(src_hbm.at[row], src_g)
        pltpu.sync_copy(dst_hbm.at[row], dst_g)

        def _ee(b):
            for k in range(BE // 16):
                sl = pl.ds(k * 16, 16)
                s16 = src_g[b, sl]
                d16 = dst_g[b, sl]
                av = plsc.load_gather(asrc_v, [s16])
                bv = plsc.load_gather(adst_v, [d16])
                xv = av + bv
                ev = jnp.where(xv >= 0.0, xv, NEG * xv)
                ee_g[b, sl] = jnp.exp(ev - mvec)
            return pltpu.async_copy(ee_g.at[b], den_sh.at[dst_g.at[b]], d0,
                                    add=True)

        dd = [None] * NBG
        gd = [None, None, None]
        sd = [None, None, None]
        dd[0] = _ee(0)
        gd[0] = pltpu.async_copy(h_hbm.at[src_g.at[0]], buf0, g0)
        dd[1] = _ee(1)
        gd[1] = pltpu.async_copy(h_hbm.at[src_g.at[1]], buf1, g1)
        for b in range(NBG):
            i = b % 3
            if b + 2 < NBG:
                dd[b + 2] = _ee(b + 2)
            gd[i].wait()
            buf = bufs[i]

            def _scale(r8, _, b=b, buf=buf):
                r0 = r8 * 8
                for dr in range(8):
                    r = r0 + dr
                    ee_s = plsc.load_gather(
                        ee_g, [jnp.full((16,), b, jnp.int32),
                               jnp.full((16,), r, jnp.int32)])
                    for q in range(8):
                        slq = pl.ds(q * 16, 16)
                        buf[r, slq] = buf[r, slq] * ee_s
                return 0
            lax.fori_loop(0, BE // 8, _scale, 0)

            sd[i] = pltpu.async_copy(buf, out_sh.at[dst_g.at[b]], ssem[i],
                                     add=True)
            nb = b + 2
            if nb < NBG:
                ni = nb % 3
                if sd[ni] is not None:
                    sd[ni].wait()
                gd[ni] = pltpu.async_copy(h_hbm.at[src_g.at[nb]], bufs[ni],
                                          gsem[ni])
        for i in range(3):
            if sd[i] is not None:
                sd[i].wait()
        for b in range(NBG):
            dd[b].wait()
        return 0
    lax.fori_loop(0, ngrp, _grp, 0)

    plsc.subcore_barrier()

    ods = []
    for t in range(RPW // 128):
        ods.append(pltpu.async_copy(out_sh.at[pl.ds(base + t * 128, 128)],
                                    out_hbm.at[cid, pl.ds(base + t * 128, 128)],
                                    g1))
    ods.append(pltpu.async_copy(den_sh.at[pl.ds(base, RPW)],
                                den_hbm.at[pl.ds(cid * ND + base, RPW)], g2))
    for od in ods:
        od.wait()


_sc_edge = functools.partial(
    pl.kernel,
    out_type=[
        jax.ShapeDtypeStruct((NC, NP, D), jnp.float32),
        jax.ShapeDtypeStruct((NC * ND,), jnp.float32),
    ],
    mesh=_mesh,
    compiler_params=pltpu.CompilerParams(needs_layout_passes=False),
    scratch_types=[
        pltpu.VMEM((NBG, BE), jnp.int32),
        pltpu.VMEM((NBG, BE), jnp.int32),
        pltpu.VMEM((NBG, BE), jnp.float32),
        pltpu.VMEM((NA,), jnp.float32),
        pltpu.VMEM((NA,), jnp.float32),
        pltpu.VMEM((128,), jnp.float32),
        pltpu.VMEM((16,), jnp.float32),
        pltpu.VMEM((BE, D), jnp.float32),
        pltpu.VMEM((BE, D), jnp.float32),
        pltpu.VMEM((BE, D), jnp.float32),
        pltpu.VMEM_SHARED((NP, D), jnp.float32),
        pltpu.VMEM_SHARED((ND,), jnp.float32),
        pltpu.SemaphoreType.DMA,
        pltpu.SemaphoreType.DMA,
        pltpu.SemaphoreType.DMA,
        pltpu.SemaphoreType.DMA,
        pltpu.SemaphoreType.DMA,
        pltpu.SemaphoreType.DMA,
        pltpu.SemaphoreType.DMA,
    ],
)(_sc_edge_body)


def _head(hin, w_ref, as_ref, ad_ref, h_ref, s_ref, d_ref, m_ref):
    h = lax.dot_general(hin, w_ref[...], (((1,), (1,)), ((), ())),
                        preferred_element_type=jnp.float32)
    h_ref[...] = h
    s = jnp.sum(h * as_ref[...], axis=1, keepdims=True)
    d = jnp.sum(h * ad_ref[...], axis=1, keepdims=True)
    zpad = jnp.zeros((NA - N, 1), jnp.float32)
    s_ref[...] = jnp.concatenate([s, zpad], axis=0)
    d_ref[...] = jnp.concatenate([d, zpad], axis=0)
    mm = jnp.max(s) + jnp.max(d)
    m_ref[...] = jnp.where(mm >= 0.0, mm, NEG * mm)[None, None]


_HEAD_OUT = [
    jax.ShapeDtypeStruct((N, D), jnp.float32),
    jax.ShapeDtypeStruct((NA, 1), jnp.float32),
    jax.ShapeDtypeStruct((NA, 1), jnp.float32),
    jax.ShapeDtypeStruct((1, 1), jnp.float32),
]


def _tc_head_body(x_ref, w_ref, as_ref, ad_ref, h_ref, s_ref, d_ref, m_ref):
    _head(x_ref[...], w_ref, as_ref, ad_ref, h_ref, s_ref, d_ref, m_ref)


_tc_head = pl.pallas_call(_tc_head_body, out_shape=_HEAD_OUT)


def _combine(p_ref, den_ref, ones_ref, b_ref):
    dd = lax.dot_general(den_ref[...], ones_ref[...], (((0,), (0,)), ((), ())),
                         preferred_element_type=jnp.float32)
    return (p_ref[0] + p_ref[1])[:N] / (dd[:N] + 1e-16) + b_ref[...]


def _tc_mid_body(p_ref, den_ref, ones_ref, b_ref, w_ref, as_ref, ad_ref,
                 h_ref, s_ref, d_ref, m_ref):
    hin = _combine(p_ref, den_ref, ones_ref, b_ref)
    _head(hin, w_ref, as_ref, ad_ref, h_ref, s_ref, d_ref, m_ref)


_tc_mid = pl.pallas_call(_tc_mid_body, out_shape=_HEAD_OUT)


def _tc_fin_body(p_ref, den_ref, ones_ref, b_ref, o_ref):
    o_ref[...] = _combine(p_ref, den_ref, ones_ref, b_ref)


_tc_fin = pl.pallas_call(
    _tc_fin_body,
    out_shape=jax.ShapeDtypeStruct((N, D), jnp.float32),
)


def kernel(x, edge_index, W1, as1, ad1, b1, W2, as2, ad2, b2):
    ei = edge_index.astype(jnp.int32)
    loop = jnp.arange(N, dtype=jnp.int32)
    pad = EP - ET
    src = jnp.concatenate([ei[0], loop, jnp.zeros((pad,), jnp.int32)])
    dst = jnp.concatenate([ei[1], loop, jnp.full((pad,), N, jnp.int32)])
    src2 = src.reshape(NW * NG, NBG, BE)
    dst2 = dst.reshape(NW * NG, NBG, BE)
    ones_col = jnp.ones((NC, 1), jnp.float32)

    h1, s1, d1, m1 = _tc_head(x, W1, as1.reshape(1, D), ad1.reshape(1, D))
    p1, den1 = _sc_edge(src2, dst2, s1.reshape(NA), d1.reshape(NA),
                        jnp.full((16,), m1[0, 0], jnp.float32), h1)

    h2, s2, d2, m2 = _tc_mid(p1, den1.reshape(NC, ND), ones_col,
                             b1.reshape(1, D), W2,
                             as2.reshape(1, D), ad2.reshape(1, D))
    p2, den2 = _sc_edge(src2, dst2, s2.reshape(NA), d2.reshape(NA),
                        jnp.full((16,), m2[0, 0], jnp.float32), h2)

    return _tc_fin(p2, den2.reshape(NC, ND), ones_col, b2.reshape(1, D))


# speedup vs baseline: 1.0528x; 1.0528x over previous
import functools

import jax
import jax.numpy as jnp
from jax import lax
from jax.experimental import pallas as pl
from jax.experimental.pallas import tpu as pltpu
from jax.experimental.pallas import tpu_sc as plsc

N = 10000
NP = 10240
ND = NP
NA = 10016
E = 320000
ET = E + N
NC = 2
NS = 16
NW = NC * NS
BE = 48
NBG = 24
NG = 9
NG0 = 10
NG1 = 8
EPW = NG * NBG * BE
EP = NW * EPW
RPW = NP // NS
NEG = 0.2
D = 128

_mesh = plsc.VectorSubcoreMesh(
    core_axis_name="c", subcore_axis_name="s", num_cores=NC, num_subcores=NS
)


def _sc_edge_body(src_hbm, dst_hbm, asrc_hbm, adst_hbm, m_hbm, h_hbm,
                  out_hbm, den_hbm,
                  src_g, dst_g, ee_g, asrc_v, adst_v, zrow, m_v,
                  buf0, buf1, buf2, out_sh, den_sh,
                  g0, g1, g2, s0, s1, s2, d0):
    cid = lax.axis_index("c")
    sid = lax.axis_index("s")
    wid = cid * NS + sid
    bufs = (buf0, buf1, buf2)
    gsem = (g0, g1, g2)
    ssem = (s0, s1, s2)

    ds0 = pltpu.async_copy(asrc_hbm, asrc_v, g0)
    ds1 = pltpu.async_copy(adst_hbm, adst_v, g1)
    ds2 = pltpu.async_copy(m_hbm, m_v, g2)

    zero16 = jnp.zeros((16,), jnp.float32)
    for q in range(8):
        zrow[pl.ds(q * 16, 16)] = zero16
    ds0.wait()
    ds1.wait()
    ds2.wait()

    def _zh(i, _):
        for q in range(8):
            buf0[i, pl.ds(q * 16, 16)] = zero16
        return 0
    lax.fori_loop(0, BE, _zh, 0)

    base = sid * RPW
    zds = []
    for t in range(13):
        zds.append(pltpu.async_copy(buf0, out_sh.at[pl.ds(base + t * BE, BE)],
                                    g1))
    zds.append(pltpu.async_copy(buf0.at[pl.ds(0, 16)],
                                out_sh.at[pl.ds(base + 624, 16)], g1))
    for t in range(RPW // 128):
        zds.append(pltpu.async_copy(zrow, den_sh.at[pl.ds(base + t * 128, 128)],
                                    g1))
    for zd in zds:
        zd.wait()
    plsc.subcore_barrier()

    mvec = m_v[...]

    ngrp = jnp.where(cid == 0, NG0, NG1)
    rowb = jnp.where(cid == 0, sid * NG0, NS * NG0 + sid * NG1)

    def _grp(g, _):
        row = rowb + g
        pltpu.sync_copy(src_hbm.at[row], src_g)
        pltpu.sync_copy(dst_hbm.at[row], dst_g)

        dd = [None] * NBG
        for b in range(NBG):
            for k in range(BE // 16):
                sl = pl.ds(k * 16, 16)
                s16 = src_g[b, sl]
                d16 = dst_g[b, sl]
                av = plsc.load_gather(asrc_v, [s16])
                bv = plsc.load_gather(adst_v, [d16])
                xv = av + bv
                ev = jnp.where(xv >= 0.0, xv, NEG * xv)
                ee_g[b, sl] = jnp.exp(ev - mvec)
            dd[b] = pltpu.async_copy(ee_g.at[b], den_sh.at[dst_g.at[b]], d0,
                                     add=True)

        gd = [None, None, None]
        sd = [None, None, None]
        gd[0] = pltpu.async_copy(h_hbm.at[src_g.at[0]], buf0, g0)
        gd[1] = pltpu.async_copy(h_hbm.at[src_g.at[1]], buf1, g1)
        for b in range(NBG):
            i = b % 3
            gd[i].wait()
            buf = bufs[i]

            def _scale(r4, _, b=b, buf=buf):
                r0 = r4 * 4
                for dr in range(4):
                    r = r0 + dr
                    ee_s = plsc.load_gather(
                        ee_g, [jnp.full((16,), b, jnp.int32),
                               jnp.full((16,), r, jnp.int32)])
                    for q in range(8):
                        slq = pl.ds(q * 16, 16)
                        buf[r, slq] = buf[r, slq] * ee_s
                return 0
            lax.fori_loop(0, BE // 4, _scale, 0)

            sd[i] = pltpu.async_copy(buf, out_sh.at[dst_g.at[b]], ssem[i],
                                     add=True)
            nb = b + 2
            if nb < NBG:
                ni = nb % 3
                if sd[ni] is not None:
                    sd[ni].wait()
                gd[ni] = pltpu.async_copy(h_hbm.at[src_g.at[nb]], bufs[ni],
                                          gsem[ni])
        for i in range(3):
            if sd[i] is not None:
                sd[i].wait()
        for b in range(NBG):
            dd[b].wait()
        return 0
    lax.fori_loop(0, ngrp, _grp, 0)

    plsc.subcore_barrier()

    ods = []
    for t in range(RPW // 128):
        ods.append(pltpu.async_copy(out_sh.at[pl.ds(base + t * 128, 128)],
                                    out_hbm.at[cid, pl.ds(base + t * 128, 128)],
                                    g1))
    ods.append(pltpu.async_copy(den_sh.at[pl.ds(base, RPW)],
                                den_hbm.at[pl.ds(cid * ND + base, RPW)], g2))
    for od in ods:
        od.wait()


_sc_edge = functools.partial(
    pl.kernel,
    out_type=[
        jax.ShapeDtypeStruct((NC, NP, D), jnp.float32),
        jax.ShapeDtypeStruct((NC * ND,), jnp.float32),
    ],
    mesh=_mesh,
    compiler_params=pltpu.CompilerParams(needs_layout_passes=False),
    scratch_types=[
        pltpu.VMEM((NBG, BE), jnp.int32),
        pltpu.VMEM((NBG, BE), jnp.int32),
        pltpu.VMEM((NBG, BE), jnp.float32),
        pltpu.VMEM((NA,), jnp.float32),
        pltpu.VMEM((NA,), jnp.float32),
        pltpu.VMEM((128,), jnp.float32),
        pltpu.VMEM((16,), jnp.float32),
        pltpu.VMEM((BE, D), jnp.float32),
        pltpu.VMEM((BE, D), jnp.float32),
        pltpu.VMEM((BE, D), jnp.float32),
        pltpu.VMEM_SHARED((NP, D), jnp.float32),
        pltpu.VMEM_SHARED((ND,), jnp.float32),
        pltpu.SemaphoreType.DMA,
        pltpu.SemaphoreType.DMA,
        pltpu.SemaphoreType.DMA,
        pltpu.SemaphoreType.DMA,
        pltpu.SemaphoreType.DMA,
        pltpu.SemaphoreType.DMA,
        pltpu.SemaphoreType.DMA,
    ],
)(_sc_edge_body)


def _head(hin, w_ref, as_ref, ad_ref, h_ref, s_ref, d_ref, m_ref):
    h = lax.dot_general(hin, w_ref[...], (((1,), (1,)), ((), ())),
                        preferred_element_type=jnp.float32)
    h_ref[...] = h
    s = jnp.sum(h * as_ref[...], axis=1, keepdims=True)
    d = jnp.sum(h * ad_ref[...], axis=1, keepdims=True)
    zpad = jnp.zeros((NA - N, 1), jnp.float32)
    s_ref[...] = jnp.concatenate([s, zpad], axis=0)
    d_ref[...] = jnp.concatenate([d, zpad], axis=0)
    mm = jnp.max(s) + jnp.max(d)
    m_ref[...] = jnp.where(mm >= 0.0, mm, NEG * mm)[None, None]


_HEAD_OUT = [
    jax.ShapeDtypeStruct((N, D), jnp.float32),
    jax.ShapeDtypeStruct((NA, 1), jnp.float32),
    jax.ShapeDtypeStruct((NA, 1), jnp.float32),
    jax.ShapeDtypeStruct((1, 1), jnp.float32),
]


def _tc_head_body(x_ref, w_ref, as_ref, ad_ref, h_ref, s_ref, d_ref, m_ref):
    _head(x_ref[...], w_ref, as_ref, ad_ref, h_ref, s_ref, d_ref, m_ref)


_tc_head = pl.pallas_call(_tc_head_body, out_shape=_HEAD_OUT)


def _combine(p_ref, den_ref, ones_ref, b_ref):
    dd = lax.dot_general(den_ref[...], ones_ref[...], (((0,), (0,)), ((), ())),
                         preferred_element_type=jnp.float32)
    return (p_ref[0] + p_ref[1])[:N] / (dd[:N] + 1e-16) + b_ref[...]


def _tc_mid_body(p_ref, den_ref, ones_ref, b_ref, w_ref, as_ref, ad_ref,
                 h_ref, s_ref, d_ref, m_ref):
    hin = _combine(p_ref, den_ref, ones_ref, b_ref)
    _head(hin, w_ref, as_ref, ad_ref, h_ref, s_ref, d_ref, m_ref)


_tc_mid = pl.pallas_call(_tc_mid_body, out_shape=_HEAD_OUT)


def _tc_fin_body(p_ref, den_ref, ones_ref, b_ref, o_ref):
    o_ref[...] = _combine(p_ref, den_ref, ones_ref, b_ref)


_tc_fin = pl.pallas_call(
    _tc_fin_body,
    out_shape=jax.ShapeDtypeStruct((N, D), jnp.float32),
)


def kernel(x, edge_index, W1, as1, ad1, b1, W2, as2, ad2, b2):
    ei = edge_index.astype(jnp.int32)
    loop = jnp.arange(N, dtype=jnp.int32)
    pad = EP - ET
    src = jnp.concatenate([ei[0], loop, jnp.zeros((pad,), jnp.int32)])
    dst = jnp.concatenate([ei[1], loop, jnp.full((pad,), N, jnp.int32)])
    src2 = src.reshape(NW * NG, NBG, BE)
    dst2 = dst.reshape(NW * NG, NBG, BE)
    ones_col = jnp.ones((NC, 1), jnp.float32)

    h1, s1, d1, m1 = _tc_head(x, W1, as1.reshape(1, D), ad1.reshape(1, D))
    p1, den1 = _sc_edge(src2, dst2, s1.reshape(NA), d1.reshape(NA),
                        jnp.full((16,), m1[0, 0], jnp.float32), h1)

    h2, s2, d2, m2 = _tc_mid(p1, den1.reshape(NC, ND), ones_col,
                             b1.reshape(1, D), W2,
                             as2.reshape(1, D), ad2.reshape(1, D))
    p2, den2 = _sc_edge(src2, dst2, s2.reshape(NA), d2.reshape(NA),
                        jnp.full((16,), m2[0, 0], jnp.float32), h2)

    return _tc_fin(p2, den2.reshape(NC, ND), ones_col, b2.reshape(1, D))
